# baseline (speedup 1.0000x reference)
import jax
import jax.numpy as jnp
from jax.experimental import pallas as pl
from jax.experimental.pallas import tpu as pltpu

N_D1 = 3
N_H2 = 3
D1_BLK = 256
H2_BLK = 1024


def _moe_kernel(x_ref, wg_ref, wn_ref, eps_ref, w1_ref, b1_ref, w2_ref, b2_ref,
                out_ref, acc_ref, w_ref, h0_ref, h1_ref, h2_ref):
    e = pl.program_id(0)
    s = pl.program_id(1)
    n_exp = wg_ref.shape[1]
    h_refs = (h0_ref, h1_ref, h2_ref)

    @pl.when((e == 0) & (s == 0))
    def _gating():
        xv = x_ref[...]
        g = jnp.dot(xv, wg_ref[...], preferred_element_type=jnp.float32)
        n = jnp.dot(xv, wn_ref[...], preferred_element_type=jnp.float32)
        logits = g + jax.nn.softplus(n) * eps_ref[...]
        lane = jax.lax.broadcasted_iota(jnp.int32, logits.shape, 1)
        i1 = jnp.argmax(logits, axis=1)[:, None]
        v1 = jnp.max(logits, axis=1)[:, None]
        oh1 = lane == i1
        masked = jnp.where(oh1, -jnp.inf, logits)
        i2 = jnp.argmax(masked, axis=1)[:, None]
        v2 = jnp.max(masked, axis=1)[:, None]
        oh2 = lane == i2
        e2 = jnp.exp(v2 - v1)
        denom = 1.0 + e2
        w_ref[...] = jnp.where(oh1, 1.0 / denom,
                               jnp.where(oh2, e2 / denom, 0.0))
        out_ref[...] = jnp.zeros_like(out_ref)

    for i in range(N_D1):
        @pl.when(s == i)
        def _w1_step(i=i):
            xc = x_ref[:, i * D1_BLK:(i + 1) * D1_BLK]
            partial = jnp.dot(xc, w1_ref[0],
                              preferred_element_type=jnp.float32)
            for j in range(N_H2):
                pj = partial[:, j * H2_BLK:(j + 1) * H2_BLK]
                if i == 0:
                    h_refs[j][...] = pj + b1_ref[0, :, j * H2_BLK:
                                                 (j + 1) * H2_BLK]
                else:
                    h_refs[j][...] += pj

    for j in range(N_H2):
        @pl.when(s == N_D1 + j)
        def _w2_step(j=j):
            hj = jnp.maximum(h_refs[j][...], 0.0)
            term = jnp.dot(hj, w2_ref[0],
                           preferred_element_type=jnp.float32)
            if j == 0:
                acc_ref[...] = term
            else:
                acc_ref[...] += term

    @pl.when(s == N_D1 + N_H2 - 1)
    def _combine():
        lane = jax.lax.broadcasted_iota(jnp.int32, (out_ref.shape[0], n_exp), 1)
        we = jnp.sum(jnp.where(lane == e, w_ref[...], 0.0), axis=1,
                     keepdims=True)
        out_ref[...] += we * (acc_ref[...] + b2_ref[0])


def kernel(x, Wg, Wnoise, W1, b1, W2, b2):
    b, c, d = x.shape
    n_exp, _, d_hid = W1.shape
    t = b * c
    x2 = x.reshape(t, d)
    eps = jax.random.normal(jax.random.key(42), (b, c, n_exp),
                            dtype=x.dtype).reshape(t, n_exp)
    out = pl.pallas_call(
        _moe_kernel,
        grid=(n_exp, N_D1 + N_H2),
        in_specs=[
            pl.BlockSpec((t, d), lambda e, s: (0, 0)),
            pl.BlockSpec((d, n_exp), lambda e, s: (0, 0)),
            pl.BlockSpec((d, n_exp), lambda e, s: (0, 0)),
            pl.BlockSpec((t, n_exp), lambda e, s: (0, 0)),
            pl.BlockSpec((1, D1_BLK, d_hid),
                         lambda e, s: (e, jnp.minimum(s, N_D1 - 1), 0)),
            pl.BlockSpec((1, 1, d_hid), lambda e, s: (e, 0, 0)),
            pl.BlockSpec((1, H2_BLK, d),
                         lambda e, s: (e, jnp.maximum(s - N_D1, 0), 0)),
            pl.BlockSpec((1, 1, d), lambda e, s: (e, 0, 0)),
        ],
        out_specs=pl.BlockSpec((t, d), lambda e, s: (0, 0)),
        out_shape=jax.ShapeDtypeStruct((t, d), x.dtype),
        scratch_shapes=[
            pltpu.VMEM((t, d), jnp.float32),
            pltpu.VMEM((t, n_exp), jnp.float32),
            pltpu.VMEM((t, H2_BLK), jnp.float32),
            pltpu.VMEM((t, H2_BLK), jnp.float32),
            pltpu.VMEM((t, H2_BLK), jnp.float32),
        ],
        compiler_params=pltpu.CompilerParams(
            dimension_semantics=("arbitrary", "arbitrary")),
    )(x2, Wg.T, Wnoise.T, eps, W1, b1[:, None, :], W2, b2[:, None, :])
    return out.reshape(b, c, d)


# speedup vs baseline: 1.4035x; 1.4035x over previous
import jax
import jax.numpy as jnp
from jax.experimental import pallas as pl
from jax.experimental.pallas import tpu as pltpu

H_BLK = 1536


def _moe_kernel(x_ref, wg_ref, wn_ref, eps_ref, w1_ref, b1_ref, w2_ref, b2_ref,
                out_ref, acc_ref, w_ref):
    e = pl.program_id(0)
    c = pl.program_id(1)
    n_chunk = pl.num_programs(1)
    n_exp = wg_ref.shape[1]

    @pl.when((e == 0) & (c == 0))
    def _gating():
        xv = x_ref[...]
        g = jnp.dot(xv, wg_ref[...], preferred_element_type=jnp.float32)
        n = jnp.dot(xv, wn_ref[...], preferred_element_type=jnp.float32)
        logits = g + jax.nn.softplus(n) * eps_ref[...]
        lane = jax.lax.broadcasted_iota(jnp.int32, logits.shape, 1)
        i1 = jnp.argmax(logits, axis=1)[:, None]
        v1 = jnp.max(logits, axis=1)[:, None]
        oh1 = lane == i1
        masked = jnp.where(oh1, -jnp.inf, logits)
        i2 = jnp.argmax(masked, axis=1)[:, None]
        v2 = jnp.max(masked, axis=1)[:, None]
        oh2 = lane == i2
        e2 = jnp.exp(v2 - v1)
        denom = 1.0 + e2
        w_ref[...] = jnp.where(oh1, 1.0 / denom,
                               jnp.where(oh2, e2 / denom, 0.0))
        out_ref[...] = jnp.zeros_like(out_ref)

    @pl.when(c == 0)
    def _init_acc():
        acc_ref[...] = jnp.zeros_like(acc_ref)

    h = jnp.dot(x_ref[...], w1_ref[0], preferred_element_type=jnp.float32)
    h = jnp.maximum(h + b1_ref[0], 0.0)
    acc_ref[...] += jnp.dot(h, w2_ref[0], preferred_element_type=jnp.float32)

    @pl.when(c == n_chunk - 1)
    def _combine():
        lane = jax.lax.broadcasted_iota(jnp.int32, (out_ref.shape[0], n_exp), 1)
        we = jnp.sum(jnp.where(lane == e, w_ref[...], 0.0), axis=1,
                     keepdims=True)
        out_ref[...] += we * (acc_ref[...] + b2_ref[0])


def kernel(x, Wg, Wnoise, W1, b1, W2, b2):
    b, c, d = x.shape
    n_exp, _, d_hid = W1.shape
    t = b * c
    x2 = x.reshape(t, d)
    eps = jax.random.normal(jax.random.key(42), (b, c, n_exp),
                            dtype=x.dtype).reshape(t, n_exp)
    n_chunk = d_hid // H_BLK
    out = pl.pallas_call(
        _moe_kernel,
        grid=(n_exp, n_chunk),
        in_specs=[
            pl.BlockSpec((t, d), lambda e, c: (0, 0)),
            pl.BlockSpec((d, n_exp), lambda e, c: (0, 0)),
            pl.BlockSpec((d, n_exp), lambda e, c: (0, 0)),
            pl.BlockSpec((t, n_exp), lambda e, c: (0, 0)),
            pl.BlockSpec((1, d, H_BLK), lambda e, c: (e, 0, c)),
            pl.BlockSpec((1, 1, H_BLK), lambda e, c: (e, 0, c)),
            pl.BlockSpec((1, H_BLK, d), lambda e, c: (e, c, 0)),
            pl.BlockSpec((1, 1, d), lambda e, c: (e, 0, 0)),
        ],
        out_specs=pl.BlockSpec((t, d), lambda e, c: (0, 0)),
        out_shape=jax.ShapeDtypeStruct((t, d), x.dtype),
        scratch_shapes=[
            pltpu.VMEM((t, d), jnp.float32),
            pltpu.VMEM((t, n_exp), jnp.float32),
        ],
        compiler_params=pltpu.CompilerParams(
            dimension_semantics=("arbitrary", "arbitrary")),
    )(x2, Wg.T, Wnoise.T, eps, W1, b1[:, None, :], W2, b2[:, None, :])
    return out.reshape(b, c, d)
